# Initial kernel scaffold; baseline (speedup 1.0000x reference)
import jax
import jax.numpy as jnp
from jax.experimental import pallas as pl


def kernel(x, Wg, We, be):
    raise NotImplementedError("write your pallas kernel here")


# speedup vs baseline: 2.0663x; 2.0663x over previous
import functools

import jax
import jax.numpy as jnp
from jax.experimental import pallas as pl
from jax.experimental.pallas import tpu as pltpu

_B, _S, _N = 32, 96, 2048
_E, _P = 8, 96
_NBLK = 512


def _moe_body(x_ref, wg_ref, wcat_ref, bet_ref, out_ref, gate_ref):
    b = pl.program_id(1)
    X = x_ref[0]

    G = jnp.dot(wg_ref[...], X, preferred_element_type=jnp.float32)
    G = G - jnp.max(G, axis=0, keepdims=True)
    Pex = jnp.exp(G)
    Pr = Pex / jnp.sum(Pex, axis=0, keepdims=True)

    @pl.when(b == 0)
    def _():
        gate_ref[...] = Pr * (1.0 / _B)

    @pl.when(b != 0)
    def _():
        gate_ref[...] += Pr * (1.0 / _B)

    iota = jax.lax.broadcasted_iota(jnp.int32, (_E, Pr.shape[1]), 0)
    m1 = jnp.max(Pr, axis=0, keepdims=True)
    i1 = jnp.min(jnp.where(Pr == m1, iota, _E), axis=0, keepdims=True)
    Pm = jnp.where(iota == i1, -jnp.inf, Pr)
    m2 = jnp.max(Pm, axis=0, keepdims=True)
    i2 = jnp.min(jnp.where(Pm == m2, iota, _E), axis=0, keepdims=True)
    w = jnp.where((iota == i1) | (iota == i2), Pr, 0.0)

    Yc = jnp.dot(wcat_ref[...], X, preferred_element_type=jnp.float32)

    acc = jnp.dot(bet_ref[...], w, preferred_element_type=jnp.float32)
    for e in range(_E):
        acc = acc + Yc[e * _P:(e + 1) * _P, :] * w[e:e + 1, :]
    out_ref[0] = acc


@functools.partial(jax.jit, static_argnames=())
def kernel(x, Wg, We, be):
    nblocks = _N // _NBLK
    wcat = We.reshape(_E * _P, _S)
    bet = be.T
    out, gate_t = pl.pallas_call(
        _moe_body,
        grid=(nblocks, _B),
        in_specs=[
            pl.BlockSpec((1, _S, _NBLK), lambda i, b: (b, 0, i)),
            pl.BlockSpec((_E, _S), lambda i, b: (0, 0)),
            pl.BlockSpec((_E * _P, _S), lambda i, b: (0, 0)),
            pl.BlockSpec((_P, _E), lambda i, b: (0, 0)),
        ],
        out_specs=[
            pl.BlockSpec((1, _P, _NBLK), lambda i, b: (b, 0, i)),
            pl.BlockSpec((_E, _NBLK), lambda i, b: (0, i)),
        ],
        out_shape=[
            jax.ShapeDtypeStruct((_B, _P, _N), jnp.float32),
            jax.ShapeDtypeStruct((_E, _N), jnp.float32),
        ],
        compiler_params=pltpu.CompilerParams(
            dimension_semantics=("parallel", "arbitrary"),
        ),
    )(x, Wg, wcat, bet)
    return out, gate_t.T


# speedup vs baseline: 3.8251x; 1.8512x over previous
import functools

import jax
import jax.numpy as jnp
from jax.experimental import pallas as pl
from jax.experimental.pallas import tpu as pltpu

_B, _S, _N = 32, 96, 2048
_E, _P = 8, 96
_NBLK = 2048


def _moe_body(x_ref, wg_ref, wcat_ref, bet_ref, out_ref, gate_ref):
    b = pl.program_id(1)
    X = x_ref[0]

    G = jnp.dot(wg_ref[...], X, preferred_element_type=jnp.float32)
    G = G - jnp.max(G, axis=0, keepdims=True)
    Pex = jnp.exp(G)
    Pr = Pex / jnp.sum(Pex, axis=0, keepdims=True)

    @pl.when(b == 0)
    def _():
        gate_ref[...] = Pr * (1.0 / _B)

    @pl.when(b != 0)
    def _():
        gate_ref[...] += Pr * (1.0 / _B)

    iota = jax.lax.broadcasted_iota(jnp.int32, (_E, Pr.shape[1]), 0)
    m1 = jnp.max(Pr, axis=0, keepdims=True)
    i1 = jnp.min(jnp.where(Pr == m1, iota, _E), axis=0, keepdims=True)
    Pm = jnp.where(iota == i1, -jnp.inf, Pr)
    m2 = jnp.max(Pm, axis=0, keepdims=True)
    i2 = jnp.min(jnp.where(Pm == m2, iota, _E), axis=0, keepdims=True)
    w = jnp.where((iota == i1) | (iota == i2), Pr, 0.0)

    Yc = jnp.dot(wcat_ref[...], X, preferred_element_type=jnp.float32)

    acc = jnp.dot(bet_ref[...], w, preferred_element_type=jnp.float32)
    for e in range(_E):
        acc = acc + Yc[e * _P:(e + 1) * _P, :] * w[e:e + 1, :]
    out_ref[0] = acc


@functools.partial(jax.jit, static_argnames=())
def kernel(x, Wg, We, be):
    nblocks = _N // _NBLK
    wcat = We.reshape(_E * _P, _S)
    bet = be.T
    out, gate_t = pl.pallas_call(
        _moe_body,
        grid=(nblocks, _B),
        in_specs=[
            pl.BlockSpec((1, _S, _NBLK), lambda i, b: (b, 0, i)),
            pl.BlockSpec((_E, _S), lambda i, b: (0, 0)),
            pl.BlockSpec((_E * _P, _S), lambda i, b: (0, 0)),
            pl.BlockSpec((_P, _E), lambda i, b: (0, 0)),
        ],
        out_specs=[
            pl.BlockSpec((1, _P, _NBLK), lambda i, b: (b, 0, i)),
            pl.BlockSpec((_E, _NBLK), lambda i, b: (0, i)),
        ],
        out_shape=[
            jax.ShapeDtypeStruct((_B, _P, _N), jnp.float32),
            jax.ShapeDtypeStruct((_E, _N), jnp.float32),
        ],
        compiler_params=pltpu.CompilerParams(
            dimension_semantics=("parallel", "arbitrary"),
        ),
    )(x, Wg, wcat, bet)
    return out, gate_t.T


# speedup vs baseline: 4.8123x; 1.2581x over previous
import functools

import jax
import jax.numpy as jnp
from jax.experimental import pallas as pl
from jax.experimental.pallas import tpu as pltpu

_B, _S, _N = 32, 96, 2048
_E, _P = 8, 96
_NBLK = 2048


def _moe_body(x_ref, wg_ref, wrow_ref, bet_ref, out_ref, gate_ref):
    b = pl.program_id(1)
    X = x_ref[0]

    G = jnp.dot(wg_ref[...], X, preferred_element_type=jnp.float32)
    G = G - jnp.max(G, axis=0, keepdims=True)
    Pex = jnp.exp(G)
    Pr = Pex / jnp.sum(Pex, axis=0, keepdims=True)

    @pl.when(b == 0)
    def _():
        gate_ref[...] = Pr * (1.0 / _B)

    @pl.when(b != 0)
    def _():
        gate_ref[...] += Pr * (1.0 / _B)

    iota = jax.lax.broadcasted_iota(jnp.int32, (_E, Pr.shape[1]), 0)
    m1 = jnp.max(Pr, axis=0, keepdims=True)
    i1 = jnp.min(jnp.where(Pr == m1, iota, _E), axis=0, keepdims=True)
    Pm = jnp.where(iota == i1, -jnp.inf, Pr)
    m2 = jnp.max(Pm, axis=0, keepdims=True)
    i2 = jnp.min(jnp.where(Pm == m2, iota, _E), axis=0, keepdims=True)
    w = jnp.where((iota == i1) | (iota == i2), Pr, 0.0)

    X_big = (w[:, None, :] * X[None, :, :]).reshape(_E * _S, X.shape[1])
    acc = jnp.dot(wrow_ref[...], X_big, preferred_element_type=jnp.float32)
    acc = acc + jnp.dot(bet_ref[...], w, preferred_element_type=jnp.float32)
    out_ref[0] = acc


@functools.partial(jax.jit, static_argnames=())
def kernel(x, Wg, We, be):
    nblocks = _N // _NBLK
    wrow = We.transpose(1, 0, 2).reshape(_P, _E * _S)
    bet = be.T
    out, gate_t = pl.pallas_call(
        _moe_body,
        grid=(nblocks, _B),
        in_specs=[
            pl.BlockSpec((1, _S, _NBLK), lambda i, b: (b, 0, i)),
            pl.BlockSpec((_E, _S), lambda i, b: (0, 0)),
            pl.BlockSpec((_P, _E * _S), lambda i, b: (0, 0)),
            pl.BlockSpec((_P, _E), lambda i, b: (0, 0)),
        ],
        out_specs=[
            pl.BlockSpec((1, _P, _NBLK), lambda i, b: (b, 0, i)),
            pl.BlockSpec((_E, _NBLK), lambda i, b: (0, i)),
        ],
        out_shape=[
            jax.ShapeDtypeStruct((_B, _P, _N), jnp.float32),
            jax.ShapeDtypeStruct((_E, _N), jnp.float32),
        ],
        compiler_params=pltpu.CompilerParams(
            dimension_semantics=("parallel", "arbitrary"),
        ),
    )(x, Wg, wrow, bet)
    return out, gate_t.T
